# baseline (speedup 1.0000x reference)
import jax
import jax.numpy as jnp
from jax import lax
from jax.experimental import pallas as pl
from jax.experimental.pallas import tpu as pltpu
from jax.experimental.pallas import tpu_sc as plsc

B = 16384
D = 128
K = D // 2

_NC = 2
_NS = 16
_NW = _NC * _NS
_ROWS_PER_W = B // _NW
_C = 128
_NCHUNK = _ROWS_PER_W // _C
_G = _C // 16


def _body(h_hbm, r_id_hbm, t_hbm, rel_hbm, out_hbm,
          idx_v, h_v, t_v, rows_v, out_v, sem):
    wid = lax.axis_index("s") * _NC + lax.axis_index("c")
    lanes = lax.iota(jnp.int32, 16)

    for chunk in range(_NCHUNK):
        base = wid * _ROWS_PER_W + chunk * _C
        pltpu.sync_copy(r_id_hbm.at[pl.ds(base, _C)], idx_v)
        gat = pltpu.async_copy(rel_hbm.at[idx_v], rows_v, sem)
        pltpu.sync_copy(h_hbm.at[pl.ds(base, _C), :], h_v)
        pltpu.sync_copy(t_hbm.at[pl.ds(base, _C), :], t_v)
        gat.wait()

        def group_body(g, _):
            rows = g * 16 + lanes

            def d_body(d, acc):
                clo = jnp.broadcast_to(d, (16,))
                chi = clo + K
                hr = plsc.load_gather(h_v, [rows, clo])
                hi = plsc.load_gather(h_v, [rows, chi])
                tr = plsc.load_gather(t_v, [rows, clo])
                ti = plsc.load_gather(t_v, [rows, chi])
                rr = plsc.load_gather(rows_v, [rows, clo])
                ri = plsc.load_gather(rows_v, [rows, chi])
                a = hr * tr + hi * ti
                b = hr * ti - hi * tr
                return acc + rr * a + ri * b

            acc = lax.fori_loop(0, K, d_body, jnp.zeros((16,), jnp.float32))
            out_v[pl.ds(g * 16, 16)] = acc
            return 0

        lax.fori_loop(0, _G, group_body, 0)
        pltpu.sync_copy(out_v, out_hbm.at[pl.ds(base, _C)])


@jax.jit
def _complex_score(h, r_id, t, rel_weight):
    mesh = plsc.VectorSubcoreMesh(core_axis_name="c", subcore_axis_name="s")
    return pl.kernel(
        _body,
        mesh=mesh,
        compiler_params=pltpu.CompilerParams(needs_layout_passes=False),
        out_type=jax.ShapeDtypeStruct((B,), jnp.float32),
        scratch_types=[
            pltpu.VMEM((_C,), jnp.int32),
            pltpu.VMEM((_C, D), jnp.float32),
            pltpu.VMEM((_C, D), jnp.float32),
            pltpu.VMEM((_C, D), jnp.float32),
            pltpu.VMEM((_C,), jnp.float32),
            pltpu.SemaphoreType.DMA,
        ],
    )(h, r_id, t, rel_weight)


def kernel(h, r_id, t, rel_weight):
    return _complex_score(h, r_id, t, rel_weight)


# speedup vs baseline: 1.0976x; 1.0976x over previous
import jax
import jax.numpy as jnp
from jax import lax
from jax.experimental import pallas as pl
from jax.experimental.pallas import tpu as pltpu
from jax.experimental.pallas import tpu_sc as plsc

B = 16384
D = 128
K = D // 2

_NC = 2
_NS = 16
_NW = _NC * _NS
_ROWS_PER_W = B // _NW
_C = 128
_NCHUNK = _ROWS_PER_W // _C
_G = _C // 16


def _body(h_hbm, r_id_hbm, t_hbm, rel_hbm, out_hbm,
          idx_v, h_v, t_v, rows_v, out_v, sem):
    wid = lax.axis_index("s") * _NC + lax.axis_index("c")
    lanes = lax.iota(jnp.int32, 16)

    def chunk_body(chunk, _):
        base = wid * _ROWS_PER_W + chunk * _C
        pltpu.sync_copy(r_id_hbm.at[pl.ds(base, _C)], idx_v)
        gat = pltpu.async_copy(rel_hbm.at[idx_v], rows_v, sem)
        pltpu.sync_copy(h_hbm.at[pl.ds(base * D, _C * D)], h_v)
        pltpu.sync_copy(t_hbm.at[pl.ds(base * D, _C * D)], t_v)
        gat.wait()

        def group_body(g, _):
            rows16 = g * 16 + lanes
            rows128 = rows16 * D

            acc = jnp.zeros((16,), jnp.float32)
            for d in range(K):
                ilo = rows128 + d
                ihi = rows128 + (d + K)
                hr = plsc.load_gather(h_v, [ilo])
                hi = plsc.load_gather(h_v, [ihi])
                tr = plsc.load_gather(t_v, [ilo])
                ti = plsc.load_gather(t_v, [ihi])
                rr = plsc.load_gather(rows_v, [rows16, jnp.full((16,), d, jnp.int32)])
                ri = plsc.load_gather(rows_v, [rows16, jnp.full((16,), d + K, jnp.int32)])
                acc = acc + rr * (hr * tr + hi * ti) + ri * (hr * ti - hi * tr)

            out_v[pl.ds(g * 16, 16)] = acc
            return 0

        lax.fori_loop(0, _G, group_body, 0)
        pltpu.sync_copy(out_v, out_hbm.at[pl.ds(base, _C)])
        return 0

    lax.fori_loop(0, _NCHUNK, chunk_body, 0)


@jax.jit
def _complex_score(h, r_id, t, rel_weight):
    mesh = plsc.VectorSubcoreMesh(core_axis_name="c", subcore_axis_name="s")
    return pl.kernel(
        _body,
        mesh=mesh,
        compiler_params=pltpu.CompilerParams(needs_layout_passes=False),
        out_type=jax.ShapeDtypeStruct((B,), jnp.float32),
        scratch_types=[
            pltpu.VMEM((_C,), jnp.int32),
            pltpu.VMEM((_C * D,), jnp.float32),
            pltpu.VMEM((_C * D,), jnp.float32),
            pltpu.VMEM((_C, D), jnp.float32),
            pltpu.VMEM((_C,), jnp.float32),
            pltpu.SemaphoreType.DMA,
        ],
    )(h.reshape(B * D), r_id, t.reshape(B * D), rel_weight)


def kernel(h, r_id, t, rel_weight):
    return _complex_score(h, r_id, t, rel_weight)


# speedup vs baseline: 3.0112x; 2.7434x over previous
import jax
import jax.numpy as jnp
from jax import lax
from jax.experimental import pallas as pl
from jax.experimental.pallas import tpu as pltpu
from jax.experimental.pallas import tpu_sc as plsc

B = 16384
D = 128
K = D // 2

_NC = 2
_NS = 16
_NW = _NC * _NS
_ROWS_PER_W = B // _NW
_C = 128
_NCHUNK = _ROWS_PER_W // _C
_G = _C // 16
_PITCH = 17


def _body(h_hbm, r_id_hbm, t_hbm, rel_hbm, out_hbm,
          idx_v, h_v, t_v, rows_v, tsc_v, out_v, sem):
    wid = lax.axis_index("s") * _NC + lax.axis_index("c")
    lanes = lax.iota(jnp.int32, 16)
    col_base = lanes * _PITCH

    def chunk_body(chunk, _):
        base = wid * _ROWS_PER_W + chunk * _C
        pltpu.sync_copy(r_id_hbm.at[pl.ds(base, _C)], idx_v)
        gat = pltpu.async_copy(rel_hbm.at[idx_v], rows_v, sem)
        pltpu.sync_copy(h_hbm.at[pl.ds(base * D, _C * D)], h_v)
        pltpu.sync_copy(t_hbm.at[pl.ds(base * D, _C * D)], t_v)
        gat.wait()

        def group_body(g, _):
            rbase = g * (16 * D)

            for i in range(16):
                row = rbase + i * D
                ridx = g * 16 + i
                acc = None
                for j in range(4):
                    lo = row + 16 * j
                    hi = lo + K
                    hr = h_v[pl.ds(lo, 16)]
                    him = h_v[pl.ds(hi, 16)]
                    tr = t_v[pl.ds(lo, 16)]
                    ti = t_v[pl.ds(hi, 16)]
                    rr = rows_v[ridx, pl.ds(16 * j, 16)]
                    ri = rows_v[ridx, pl.ds(K + 16 * j, 16)]
                    term = rr * (hr * tr + him * ti) + ri * (hr * ti - him * tr)
                    acc = term if acc is None else acc + term
                tsc_v[pl.ds(i * _PITCH, 16)] = acc

            tot = plsc.load_gather(tsc_v, [col_base])
            for j in range(1, 16):
                tot = tot + plsc.load_gather(tsc_v, [col_base + j])
            out_v[pl.ds(g * 16, 16)] = tot
            return 0

        lax.fori_loop(0, _G, group_body, 0)
        pltpu.sync_copy(out_v, out_hbm.at[pl.ds(base, _C)])
        return 0

    lax.fori_loop(0, _NCHUNK, chunk_body, 0)


@jax.jit
def _complex_score(h, r_id, t, rel_weight):
    mesh = plsc.VectorSubcoreMesh(core_axis_name="c", subcore_axis_name="s")
    return pl.kernel(
        _body,
        mesh=mesh,
        compiler_params=pltpu.CompilerParams(needs_layout_passes=False),
        out_type=jax.ShapeDtypeStruct((B,), jnp.float32),
        scratch_types=[
            pltpu.VMEM((_C,), jnp.int32),
            pltpu.VMEM((_C * D,), jnp.float32),
            pltpu.VMEM((_C * D,), jnp.float32),
            pltpu.VMEM((_C, D), jnp.float32),
            pltpu.VMEM((16 * _PITCH,), jnp.float32),
            pltpu.VMEM((_C,), jnp.float32),
            pltpu.SemaphoreType.DMA,
        ],
    )(h.reshape(B * D), r_id, t.reshape(B * D), rel_weight)


def kernel(h, r_id, t, rel_weight):
    return _complex_score(h, r_id, t, rel_weight)


# speedup vs baseline: 3.4678x; 1.1517x over previous
import jax
import jax.numpy as jnp
from jax import lax
from jax.experimental import pallas as pl
from jax.experimental.pallas import tpu as pltpu
from jax.experimental.pallas import tpu_sc as plsc

B = 16384
D = 128
K = D // 2

_NC = 2
_NS = 16
_NW = _NC * _NS
_ROWS_PER_W = B // _NW
_C = 128
_NCHUNK = _ROWS_PER_W // _C
_G = _C // 16
_PITCH = 17


def _body(h_hbm, r_id_hbm, t_hbm, rel_hbm, out_hbm,
          idx_all, out_all, tsc_v,
          h_v0, t_v0, rows_v0, h_v1, t_v1, rows_v1,
          sem_h0, sem_t0, sem_r0, sem_h1, sem_t1, sem_r1):
    wid = lax.axis_index("s") * _NC + lax.axis_index("c")
    lanes = lax.iota(jnp.int32, 16)
    col_base = lanes * _PITCH
    base = wid * _ROWS_PER_W

    pltpu.sync_copy(r_id_hbm.at[pl.ds(base, _ROWS_PER_W)], idx_all)

    bufs = ((h_v0, t_v0, rows_v0, sem_h0, sem_t0, sem_r0),
            (h_v1, t_v1, rows_v1, sem_h1, sem_t1, sem_r1))

    def issue(c):
        h_v, t_v, rows_v, sem_h, sem_t, sem_r = bufs[c % 2]
        cbase = base + c * _C
        hr_ = pltpu.async_copy(
            rel_hbm.at[idx_all.at[pl.ds(c * _C, _C)]], rows_v, sem_r)
        hh = pltpu.async_copy(h_hbm.at[pl.ds(cbase * D, _C * D)], h_v, sem_h)
        ht = pltpu.async_copy(t_hbm.at[pl.ds(cbase * D, _C * D)], t_v, sem_t)
        return hh, ht, hr_

    handles = issue(0)
    for c in range(_NCHUNK):
        h_v, t_v, rows_v = bufs[c % 2][:3]
        nxt = issue(c + 1) if c + 1 < _NCHUNK else None
        for hd in handles:
            hd.wait()
        handles = nxt

        def group_body(g, _):
            rbase = g * (16 * D)

            for i in range(16):
                row = rbase + i * D
                ridx = g * 16 + i
                acc = None
                for j in range(4):
                    lo = row + 16 * j
                    hi = lo + K
                    hr = h_v[pl.ds(lo, 16)]
                    him = h_v[pl.ds(hi, 16)]
                    tr = t_v[pl.ds(lo, 16)]
                    ti = t_v[pl.ds(hi, 16)]
                    rr = rows_v[ridx, pl.ds(16 * j, 16)]
                    ri = rows_v[ridx, pl.ds(K + 16 * j, 16)]
                    term = rr * (hr * tr + him * ti) + ri * (hr * ti - him * tr)
                    acc = term if acc is None else acc + term
                tsc_v[pl.ds(i * _PITCH, 16)] = acc

            tot = plsc.load_gather(tsc_v, [col_base])
            for j in range(1, 16):
                tot = tot + plsc.load_gather(tsc_v, [col_base + j])
            out_all[pl.ds(c * _C + g * 16, 16)] = tot
            return 0

        lax.fori_loop(0, _G, group_body, 0)

    pltpu.sync_copy(out_all, out_hbm.at[pl.ds(base, _ROWS_PER_W)])


@jax.jit
def _complex_score(h, r_id, t, rel_weight):
    mesh = plsc.VectorSubcoreMesh(core_axis_name="c", subcore_axis_name="s")
    return pl.kernel(
        _body,
        mesh=mesh,
        compiler_params=pltpu.CompilerParams(needs_layout_passes=False),
        out_type=jax.ShapeDtypeStruct((B,), jnp.float32),
        scratch_types=[
            pltpu.VMEM((_ROWS_PER_W,), jnp.int32),
            pltpu.VMEM((_ROWS_PER_W,), jnp.float32),
            pltpu.VMEM((16 * _PITCH,), jnp.float32),
            pltpu.VMEM((_C * D,), jnp.float32),
            pltpu.VMEM((_C * D,), jnp.float32),
            pltpu.VMEM((_C, D), jnp.float32),
            pltpu.VMEM((_C * D,), jnp.float32),
            pltpu.VMEM((_C * D,), jnp.float32),
            pltpu.VMEM((_C, D), jnp.float32),
            pltpu.SemaphoreType.DMA,
            pltpu.SemaphoreType.DMA,
            pltpu.SemaphoreType.DMA,
            pltpu.SemaphoreType.DMA,
            pltpu.SemaphoreType.DMA,
            pltpu.SemaphoreType.DMA,
        ],
    )(h.reshape(B * D), r_id, t.reshape(B * D), rel_weight)


def kernel(h, r_id, t, rel_weight):
    return _complex_score(h, r_id, t, rel_weight)


# speedup vs baseline: 3.9350x; 1.1347x over previous
import jax
import jax.numpy as jnp
from jax import lax
from jax.experimental import pallas as pl
from jax.experimental.pallas import tpu as pltpu
from jax.experimental.pallas import tpu_sc as plsc

B = 16384
D = 128
K = D // 2

_NC = 2
_NS = 16
_NW = _NC * _NS
_ROWS_PER_W = B // _NW
_C = 128
_NCHUNK = _ROWS_PER_W // _C
_G = _C // 16
_PITCH = 17


def _body(h_hbm, r_id_hbm, t_hbm, rel_hbm, out_hbm,
          idx_all, out_all, tsc_v,
          h_v0, t_v0, rows_v0, h_v1, t_v1, rows_v1,
          sem_h0, sem_t0, sem_r0, sem_h1, sem_t1, sem_r1):
    wid = lax.axis_index("s") * _NC + lax.axis_index("c")
    lanes = lax.iota(jnp.int32, 16)
    base = wid * _ROWS_PER_W

    pltpu.sync_copy(r_id_hbm.at[pl.ds(base, _ROWS_PER_W)], idx_all)

    bufs = ((h_v0, t_v0, rows_v0, sem_h0, sem_t0, sem_r0),
            (h_v1, t_v1, rows_v1, sem_h1, sem_t1, sem_r1))

    def issue(c):
        h_v, t_v, rows_v, sem_h, sem_t, sem_r = bufs[c % 2]
        cbase = base + c * _C
        hr_ = pltpu.async_copy(
            rel_hbm.at[idx_all.at[pl.ds(c * _C, _C)]], rows_v, sem_r)
        hh = pltpu.async_copy(h_hbm.at[pl.ds(cbase * D, _C * D)], h_v, sem_h)
        ht = pltpu.async_copy(t_hbm.at[pl.ds(cbase * D, _C * D)], t_v, sem_t)
        return hh, ht, hr_

    handles = issue(0)
    for c in range(_NCHUNK):
        h_v, t_v, rows_v = bufs[c % 2][:3]
        nxt = issue(c + 1) if c + 1 < _NCHUNK else None
        for hd in handles:
            hd.wait()
        handles = nxt

        @plsc.parallel_loop(0, _C, unroll=4)
        def row_body(r):
            row = r * D
            acc1 = None
            acc2 = None
            for j in range(4):
                lo = row + 16 * j
                hi = lo + K
                hr = h_v[pl.ds(lo, 16)]
                him = h_v[pl.ds(hi, 16)]
                tr = t_v[pl.ds(lo, 16)]
                ti = t_v[pl.ds(hi, 16)]
                rr = rows_v[r, pl.ds(16 * j, 16)]
                ri = rows_v[r, pl.ds(K + 16 * j, 16)]
                t1 = rr * (hr * tr + him * ti)
                t2 = ri * (hr * ti - him * tr)
                acc1 = t1 if acc1 is None else acc1 + t1
                acc2 = t2 if acc2 is None else acc2 + t2
            tsc_v[pl.ds(r * _PITCH, 16)] = acc1 + acc2

        @plsc.parallel_loop(0, _G, unroll=2)
        def red_body(g):
            col_base = (g * 16 + lanes) * _PITCH
            parts = [plsc.load_gather(tsc_v, [col_base + j]) for j in range(16)]
            while len(parts) > 1:
                parts = [a + b for a, b in zip(parts[::2], parts[1::2])]
            out_all[pl.ds(c * _C + g * 16, 16)] = parts[0]

    pltpu.sync_copy(out_all, out_hbm.at[pl.ds(base, _ROWS_PER_W)])


@jax.jit
def _complex_score(h, r_id, t, rel_weight):
    mesh = plsc.VectorSubcoreMesh(core_axis_name="c", subcore_axis_name="s")
    return pl.kernel(
        _body,
        mesh=mesh,
        compiler_params=pltpu.CompilerParams(needs_layout_passes=False),
        out_type=jax.ShapeDtypeStruct((B,), jnp.float32),
        scratch_types=[
            pltpu.VMEM((_ROWS_PER_W,), jnp.int32),
            pltpu.VMEM((_ROWS_PER_W,), jnp.float32),
            pltpu.VMEM((_C * _PITCH,), jnp.float32),
            pltpu.VMEM((_C * D,), jnp.float32),
            pltpu.VMEM((_C * D,), jnp.float32),
            pltpu.VMEM((_C, D), jnp.float32),
            pltpu.VMEM((_C * D,), jnp.float32),
            pltpu.VMEM((_C * D,), jnp.float32),
            pltpu.VMEM((_C, D), jnp.float32),
            pltpu.SemaphoreType.DMA,
            pltpu.SemaphoreType.DMA,
            pltpu.SemaphoreType.DMA,
            pltpu.SemaphoreType.DMA,
            pltpu.SemaphoreType.DMA,
            pltpu.SemaphoreType.DMA,
        ],
    )(h.reshape(B * D), r_id, t.reshape(B * D), rel_weight)


def kernel(h, r_id, t, rel_weight):
    return _complex_score(h, r_id, t, rel_weight)
